# baseline (speedup 1.0000x reference)
import functools

import jax
import jax.numpy as jnp
from jax import lax
from jax.experimental import pallas as pl
from jax.experimental.pallas import tpu as pltpu
from jax.experimental.pallas import tpu_sc as plsc

N = 10000
E = 320000
D = 128
H = 32
OUT = 32
T = 20

_NC = 2
_NS = 16
_NW = _NC * _NS
_W = 48
_C = 400
_EPW = E // _NW
_NCH = _EPW // _C
_NB = 4
_RPS = 1000
_NZ = N // _RPS


def _proj_body(nf_ref, wself_ref, wneigh_ref, wcls_ref, ps_ref, pn_ref):
    wcls = wcls_ref[...]
    wsc = lax.dot_general(wcls, wself_ref[...], (((1,), (0,)), ((), ())),
                          preferred_element_type=jnp.float32)
    wnc = lax.dot_general(wcls, wneigh_ref[...], (((1,), (0,)), ((), ())),
                          preferred_element_type=jnp.float32)
    x = nf_ref[...]
    ps_ref[...] = lax.dot_general(x, wsc, (((1,), (1,)), ((), ())),
                                  preferred_element_type=jnp.float32)
    pn = lax.dot_general(x, wnc, (((1,), (1,)), ((), ())),
                         preferred_element_type=jnp.float32)
    bn = pn.shape[0]
    pn_ref[...] = jnp.concatenate(
        [pn, jnp.ones((bn, 1), jnp.float32), jnp.zeros((bn, _W - OUT - 1), jnp.float32)],
        axis=1)


def _lstm_body(hist_ref, wih_ref, whh_ref, b_ref, wcls_ref, out_ref):
    bn = hist_ref.shape[0]
    wih = wih_ref[...]
    whh = whh_ref[...]
    b = b_ref[...]
    h = jnp.zeros((bn, H), jnp.float32)
    c = jnp.zeros((bn, H), jnp.float32)
    for t in range(T):
        x_t = hist_ref[:, t, :]
        gates = (lax.dot_general(x_t, wih, (((1,), (1,)), ((), ())),
                                 preferred_element_type=jnp.float32)
                 + lax.dot_general(h, whh, (((1,), (1,)), ((), ())),
                                   preferred_element_type=jnp.float32)
                 + b)
        i = jax.nn.sigmoid(gates[:, :H])
        f = jax.nn.sigmoid(gates[:, H:2 * H])
        g = jnp.tanh(gates[:, 2 * H:3 * H])
        o = jax.nn.sigmoid(gates[:, 3 * H:])
        c = f * c + i * g
        h = o * jnp.tanh(c)
    out_ref[...] = lax.dot_general(h, wcls_ref[...], (((1,), (1,)), ((), ())),
                                   preferred_element_type=jnp.float32)


def _combine_body(ps_ref, ho_ref, acc_ref, bsage_ref, wcls_ref,
                  bcls_ref, out_ref):
    acc = acc_ref[0, :, :OUT] + acc_ref[1, :, :OUT]
    deg = acc_ref[0, :, OUT:OUT + 1] + acc_ref[1, :, OUT:OUT + 1]
    neigh = acc / jnp.maximum(deg, 1.0)
    bsc = lax.dot_general(bsage_ref[...], wcls_ref[...], (((1,), (1,)), ((), ())),
                          preferred_element_type=jnp.float32)
    out_ref[...] = ps_ref[...] + neigh + ho_ref[...] + bsc + bcls_ref[...]


def _sc_agg_body(pn_hbm, src_hbm, dst_hbm, z_hbm, acc_out,
                 srcb, dstb, r0, r1, r2, r3, acc_sh,
                 gs0, gs1, gs2, gs3, ss0, ss1, ss2, ss3):
    c = lax.axis_index("c")
    s = lax.axis_index("s")
    wid = s * _NC + c
    rows = (r0, r1, r2, r3)
    gsem = (gs0, gs1, gs2, gs3)
    ssem = (ss0, ss1, ss2, ss3)

    @pl.when(s < _NZ)
    def _zero():
        pltpu.sync_copy(z_hbm.at[pl.ds(s * _RPS, _RPS)],
                        acc_sh.at[pl.ds(s * _RPS, _RPS)])

    base = wid * _EPW
    pltpu.sync_copy(src_hbm.at[pl.ds(base, _EPW)], srcb)
    pltpu.sync_copy(dst_hbm.at[pl.ds(base, _EPW)], dstb)
    plsc.subcore_barrier()

    def span(g):
        return pl.ds(g * _C, _C)

    gd = {}
    sd = {}
    gd[0] = pltpu.async_copy(pn_hbm.at[srcb.at[span(0)]], rows[0], gsem[0])
    gd[1] = pltpu.async_copy(pn_hbm.at[srcb.at[span(1)]], rows[1], gsem[1])
    for g in range(_NCH):
        b = g % _NB
        if g + 2 < _NCH:
            bb = (g + 2) % _NB
            if g >= 2:
                sd[g - 2].wait()
            gd[g + 2] = pltpu.async_copy(pn_hbm.at[srcb.at[span(g + 2)]],
                                         rows[bb], gsem[bb])
        gd[g].wait()
        sd[g] = pltpu.async_copy(rows[b], acc_sh.at[dstb.at[span(g)]],
                                 ssem[b], add=True)
    for g in range(_NCH - 4, _NCH):
        sd[g].wait()
    plsc.subcore_barrier()

    @pl.when(s < _NZ)
    def _writeback():
        pltpu.sync_copy(acc_sh.at[pl.ds(s * _RPS, _RPS)],
                        acc_out.at[c, pl.ds(s * _RPS, _RPS)])


def _sc_aggregate(pn, src, dst):
    z = jnp.zeros((N, _W), jnp.float32)
    mesh = plsc.VectorSubcoreMesh(core_axis_name="c", subcore_axis_name="s")
    f = pl.kernel(
        _sc_agg_body,
        out_type=jax.ShapeDtypeStruct((_NC, N, _W), jnp.float32),
        mesh=mesh,
        scratch_types=[
            pltpu.VMEM((_EPW,), jnp.int32),
            pltpu.VMEM((_EPW,), jnp.int32),
            pltpu.VMEM((_C, _W), jnp.float32),
            pltpu.VMEM((_C, _W), jnp.float32),
            pltpu.VMEM((_C, _W), jnp.float32),
            pltpu.VMEM((_C, _W), jnp.float32),
            pltpu.VMEM_SHARED((N, _W), jnp.float32),
            pltpu.SemaphoreType.DMA,
            pltpu.SemaphoreType.DMA,
            pltpu.SemaphoreType.DMA,
            pltpu.SemaphoreType.DMA,
            pltpu.SemaphoreType.DMA,
            pltpu.SemaphoreType.DMA,
            pltpu.SemaphoreType.DMA,
            pltpu.SemaphoreType.DMA,
        ],
        compiler_params=pltpu.CompilerParams(use_tc_tiling_on_sc=False),
    )
    return f(pn, src, dst, z)


_BN = 400


def _tc_proj(node_feats, W_self, W_neigh, W_cls):
    grid = (N // _BN,)
    return pl.pallas_call(
        _proj_body,
        grid=grid,
        in_specs=[
            pl.BlockSpec((_BN, D), lambda i: (i, 0)),
            pl.BlockSpec((H, D), lambda i: (0, 0)),
            pl.BlockSpec((H, D), lambda i: (0, 0)),
            pl.BlockSpec((OUT, H), lambda i: (0, 0)),
        ],
        out_specs=[
            pl.BlockSpec((_BN, OUT), lambda i: (i, 0)),
            pl.BlockSpec((_BN, _W), lambda i: (i, 0)),
        ],
        out_shape=[
            jax.ShapeDtypeStruct((N, OUT), jnp.float32),
            jax.ShapeDtypeStruct((N, _W), jnp.float32),
        ],
    )(node_feats, W_self, W_neigh, W_cls)


def _tc_lstm(history_feats, W_ih, W_hh, b, W_cls):
    grid = (N // _BN,)
    return pl.pallas_call(
        _lstm_body,
        grid=grid,
        in_specs=[
            pl.BlockSpec((_BN, T, H), lambda i: (i, 0, 0)),
            pl.BlockSpec((4 * H, H), lambda i: (0, 0)),
            pl.BlockSpec((4 * H, H), lambda i: (0, 0)),
            pl.BlockSpec((1, 4 * H), lambda i: (0, 0)),
            pl.BlockSpec((OUT, H), lambda i: (0, 0)),
        ],
        out_specs=pl.BlockSpec((_BN, OUT), lambda i: (i, 0)),
        out_shape=jax.ShapeDtypeStruct((N, OUT), jnp.float32),
    )(history_feats, W_ih, W_hh, b, W_cls)


def _tc_combine(ps, ho, acc, b_sage, W_cls, b_cls):
    grid = (N // _BN,)
    return pl.pallas_call(
        _combine_body,
        grid=grid,
        in_specs=[
            pl.BlockSpec((_BN, OUT), lambda i: (i, 0)),
            pl.BlockSpec((_BN, OUT), lambda i: (i, 0)),
            pl.BlockSpec((_NC, _BN, _W), lambda i: (0, i, 0)),
            pl.BlockSpec((1, H), lambda i: (0, 0)),
            pl.BlockSpec((OUT, H), lambda i: (0, 0)),
            pl.BlockSpec((1, OUT), lambda i: (0, 0)),
        ],
        out_specs=pl.BlockSpec((_BN, OUT), lambda i: (i, 0)),
        out_shape=jax.ShapeDtypeStruct((N, OUT), jnp.float32),
    )(ps, ho, acc, b_sage, W_cls, b_cls)


def kernel(node_feats, edge_index, history_feats, W_self, W_neigh, b_sage,
           W_ih, W_hh, b_ih, b_hh, W_cls, b_cls):
    src = edge_index[0].astype(jnp.int32)
    dst = edge_index[1].astype(jnp.int32)
    b = (b_ih + b_hh).reshape(1, 4 * H)

    ps, pn = _tc_proj(node_feats, W_self, W_neigh, W_cls)
    acc = _sc_aggregate(pn, src, dst)
    ho = _tc_lstm(history_feats, W_ih, W_hh, b, W_cls)
    return _tc_combine(ps, ho, acc, b_sage.reshape(1, H), W_cls,
                       b_cls.reshape(1, OUT))
